# baseline (speedup 1.0000x reference)
import jax
import jax.numpy as jnp
from jax.experimental import pallas as pl
from jax.experimental.pallas import tpu as pltpu

_KB = 1024
_W = 128


def _argmin_body(x_ref, y_ref, out_ref, yt_ref, rdist_ref, ridx_ref):
    kt = pl.program_id(0)

    @pl.when(kt == 0)
    def _init():
        yt_ref[...] = y_ref[...].T
        rdist_ref[...] = jnp.full(rdist_ref.shape, jnp.inf, jnp.float32)
        ridx_ref[...] = jnp.zeros(ridx_ref.shape, jnp.int32)

    x = x_ref[...]
    rd = rdist_ref[...]
    ri = ridx_ref[...]
    lane = jax.lax.broadcasted_iota(jnp.int32, rd.shape, 1)
    for j in range(_KB // _W):
        sj = jax.lax.dot_general(
            x, yt_ref[:, pl.ds(kt * _KB + j * _W, _W)], (((1,), (0,)), ((), ())),
            preferred_element_type=jnp.float32,
        )
        dj = 1.0 - sj
        colj = lane + (kt * _KB + j * _W)
        mask = dj < rd
        rd = jnp.minimum(rd, dj)
        ri = jnp.where(mask, colj, ri)
    rdist_ref[...] = rd
    ridx_ref[...] = ri

    @pl.when(kt == pl.num_programs(0) - 1)
    def _finish():
        m = jnp.min(rd, axis=1, keepdims=True)
        cand = jnp.where(rd == m, ri, jnp.int32(2**31 - 1))
        out_ref[...] = jnp.min(cand, axis=1, keepdims=True).reshape(1, -1)


def kernel(desc, anchors):
    B, D = desc.shape
    K, _ = anchors.shape

    xn = jnp.linalg.norm(desc, axis=-1, keepdims=True)
    x = desc / jnp.maximum(xn, 1e-12)
    yn = jnp.linalg.norm(anchors, axis=-1, keepdims=True)
    y = anchors / jnp.maximum(yn, 1e-12)

    ids = pl.pallas_call(
        _argmin_body,
        grid=(K // _KB,),
        in_specs=[
            pl.BlockSpec((B, D), lambda k: (0, 0)),
            pl.BlockSpec((K, D), lambda k: (0, 0)),
        ],
        out_specs=pl.BlockSpec((1, B), lambda k: (0, 0)),
        out_shape=jax.ShapeDtypeStruct((1, B), jnp.int32),
        scratch_shapes=[
            pltpu.VMEM((D, K), jnp.float32),
            pltpu.VMEM((B, _W), jnp.float32),
            pltpu.VMEM((B, _W), jnp.int32),
        ],
        compiler_params=pltpu.CompilerParams(
            dimension_semantics=("arbitrary",),
        ),
    )(x, y)
    return ids.reshape(B)


# speedup vs baseline: 1.2074x; 1.2074x over previous
import jax
import jax.numpy as jnp
from jax.experimental import pallas as pl
from jax.experimental.pallas import tpu as pltpu

_KB = 1024
_W = 128


def _argmin_body(x_ref, yt_ref, out_ref, rbest_ref, ridx_ref):
    kt = pl.program_id(0)

    @pl.when(kt == 0)
    def _init():
        rbest_ref[...] = jnp.full(rbest_ref.shape, -jnp.inf, jnp.float32)
        ridx_ref[...] = jnp.zeros(ridx_ref.shape, jnp.int32)

    x = x_ref[...]
    rd = rbest_ref[...]
    ri = ridx_ref[...]
    lane = jax.lax.broadcasted_iota(jnp.int32, rd.shape, 1)
    for j in range(_KB // _W):
        sj = jax.lax.dot_general(
            x, yt_ref[:, j * _W:(j + 1) * _W], (((1,), (0,)), ((), ())),
            preferred_element_type=jnp.float32,
        )
        colj = lane + (kt * _KB + j * _W)
        mask = sj > rd
        rd = jnp.maximum(rd, sj)
        ri = jnp.where(mask, colj, ri)
    rbest_ref[...] = rd
    ridx_ref[...] = ri

    @pl.when(kt == pl.num_programs(0) - 1)
    def _finish():
        dw = 1.0 - rd
        m = jnp.min(dw, axis=1, keepdims=True)
        cand = jnp.where(dw == m, ri, jnp.int32(2**31 - 1))
        out_ref[...] = jnp.min(cand, axis=1, keepdims=True).reshape(1, -1)


def kernel(desc, anchors):
    B, D = desc.shape
    K, _ = anchors.shape

    xn = jnp.linalg.norm(desc, axis=-1, keepdims=True)
    x = desc / jnp.maximum(xn, 1e-12)
    yn = jnp.linalg.norm(anchors, axis=-1, keepdims=True)
    y = anchors / jnp.maximum(yn, 1e-12)
    yt = y.T

    ids = pl.pallas_call(
        _argmin_body,
        grid=(K // _KB,),
        in_specs=[
            pl.BlockSpec((B, D), lambda k: (0, 0)),
            pl.BlockSpec((D, _KB), lambda k: (0, k)),
        ],
        out_specs=pl.BlockSpec((1, B), lambda k: (0, 0)),
        out_shape=jax.ShapeDtypeStruct((1, B), jnp.int32),
        scratch_shapes=[
            pltpu.VMEM((B, _W), jnp.float32),
            pltpu.VMEM((B, _W), jnp.int32),
        ],
        compiler_params=pltpu.CompilerParams(
            dimension_semantics=("arbitrary",),
        ),
    )(x, yt)
    return ids.reshape(B)
